# baseline (speedup 1.0000x reference)
import jax
import jax.numpy as jnp
from jax import lax
from jax.experimental import pallas as pl
from jax.experimental.pallas import tpu as pltpu
from jax.experimental.pallas import tpu_sc as plsc

VOCAB_SIZE = 1_000_000
N_EMBD = 64
SEQ_LEN = 200
BATCH = 1024
NTOK = BATCH * SEQ_LEN

_info = plsc.get_sparse_core_info()
_NC, _NS = _info.num_cores, _info.num_subcores
NW = _NC * _NS
SLAB = 128
NSLAB = 246
STRIPE = NSLAB * SLAB
CAP = 8192
NCH = NTOK // 1024
DUMP = NTOK
NR = 4


def _emb_body(xi_hbm, ttT_hbm, pos_hbm, tailT_hbm, out_hbm,
              xs, r0, t0, r1, t1, slab_v, pos_v, tail_v, obuf, tl_v, bnd,
              xsem, slsem, scsem):
    cid = lax.axis_index("c")
    sid = lax.axis_index("s")
    wid = sid * _NC + cid
    lo = wid * STRIPE
    hi = lo + STRIPE

    pltpu.sync_copy(pos_hbm, pos_v)

    iota = lax.iota(jnp.int32, 16)

    def xstage(ch, b):
        off = pl.multiple_of(ch * 8, 8)
        pltpu.async_copy(xi_hbm.at[pl.ds(off, 8)], xs.at[b], xsem.at[b])

    def xwait(b):
        pltpu.make_async_copy(xi_hbm.at[pl.ds(0, 8)], xs.at[b],
                              xsem.at[b]).wait()

    xstage(0, 0)
    xstage(1, 1)

    def scan_chunk(ch, cur):
        for b in range(2):
            c = ch * 2 + b
            xwait(b)
            for rr in range(8):
                for g in range(8):
                    v = xs[b, rr, pl.ds(g * 16, 16)]
                    m = (v >= lo) & (v < hi)
                    tvec = (c * 1024 + rr * 128 + g * 16) + iota
                    plsc.store_compressed(r0.at[pl.ds(cur, 16)], v - lo, mask=m)
                    plsc.store_compressed(t0.at[pl.ds(cur, 16)], tvec, mask=m)
                    cnt = plsc.all_reduce_population_count(m)
                    cur = cur + cnt[0]

            @pl.when(ch * 2 + b + 2 < NCH)
            def _():
                xstage(c + 2, b)
        return cur

    nmatch = lax.fori_loop(0, NCH // 2, scan_chunk, 0)
    bnd[0, 0] = 0
    bnd[0, 1] = nmatch

    bufs = ((r0, t0), (r1, t1))
    for lev in range(8):
        src, dst = bufs[lev % 2], bufs[(lev + 1) % 2]
        pin, pout = lev % 2, (lev + 1) % 2
        bit = 7 + (7 - lev)

        def seg_body(j, c, src=src, dst=dst, pin=pin, pout=pout, bit=bit):
            start = bnd[pin, j]
            end = bnd[pin, j + 1]

            def grp(g, cs):
                c0, c1 = cs
                p = start + g * 16
                rv = src[0][pl.ds(p, 16)]
                tv = src[1][pl.ds(p, 16)]
                valid = (p + iota) < end
                one = ((rv >> bit) & 1) == 1
                m0 = valid & (~one)
                m1 = valid & one
                n0 = plsc.all_reduce_population_count(m0)[0]
                n1 = plsc.all_reduce_population_count(m1)[0]
                plsc.store_compressed(dst[0].at[pl.ds(c0, 16)], rv, mask=m0)
                plsc.store_compressed(dst[1].at[pl.ds(c0, 16)], tv, mask=m0)
                rvr = lax.rev(rv, (0,))
                tvr = lax.rev(tv, (0,))
                validr = (p + 15 - iota) < end
                m1r = validr & ((((rvr >> bit) & 1)) == 1)
                plsc.store_compressed(dst[0].at[pl.ds(c1 - n1, 16)], rvr, mask=m1r)
                plsc.store_compressed(dst[1].at[pl.ds(c1 - n1, 16)], tvr, mask=m1r)
                return (c0 + n0, c1 - n1)

            ngrp = (end - start + 15) >> 4
            mid, _ = lax.fori_loop(0, ngrp, grp, (start, end))
            bnd[pout, 2 * j] = start
            bnd[pout, 2 * j + 1] = mid
            return c

        lax.fori_loop(0, 1 << lev, seg_body, 0)
        bnd[(lev + 1) % 2, 2 << lev] = nmatch

    def slstage(s):
        cb_raw = lo + s * SLAB
        cb = jnp.where(cb_raw + SLAB > VOCAB_SIZE, 0, cb_raw)
        cb = pl.multiple_of(cb, SLAB)
        rb = lax.rem(s, NR)
        pltpu.async_copy(ttT_hbm.at[:, pl.ds(cb, SLAB)], slab_v.at[rb],
                         slsem.at[rb])

    def slwait(rb):
        pltpu.make_async_copy(ttT_hbm.at[:, pl.ds(0, SLAB)], slab_v.at[rb],
                              slsem.at[rb]).wait()

    def scat_wait(rb):
        pltpu.make_async_copy(obuf.at[rb], out_hbm.at[pl.ds(0, 16)],
                              scsem.at[rb]).wait()

    pltpu.sync_copy(tailT_hbm, tail_v)

    dumpvec = jnp.full((16,), DUMP, jnp.int32)
    for bb in range(NR):
        tl_v[bb, pl.ds(0, 16)] = dumpvec
        pltpu.async_copy(obuf.at[bb], out_hbm.at[tl_v.at[bb]], scsem.at[bb])

    for s0 in range(NR):
        slstage(s0)

    def emit_groups(src_ref, coff, start, end, ncol):
        def grp(g, c):
            p = start + g * 16
            rv = r0[pl.ds(p, 16)]
            tv = t0[pl.ds(p, 16)]
            valid = (p + iota) < end
            cvec = jnp.clip(rv - coff, 0, ncol - 1)
            lrow = jnp.clip(lax.rem(tv, SEQ_LEN), 0, SEQ_LEN - 1)
            rb = lax.rem(g, NR)
            scat_wait(rb)
            for d in range(N_EMBD):
                dfull = jnp.full((16,), d, jnp.int32)
                vals = plsc.load_gather(src_ref, [dfull, cvec])
                pvals = plsc.load_gather(pos_v, [lrow, dfull])
                plsc.store_scatter(obuf.at[rb], [iota, dfull], vals + pvals)
            tl_v[rb, pl.ds(0, 16)] = jnp.where(valid, tv, DUMP)
            pltpu.async_copy(obuf.at[rb], out_hbm.at[tl_v.at[rb]],
                             scsem.at[rb])
            return c

        ngrp = (end - start + 15) >> 4
        lax.fori_loop(0, ngrp, grp, 0)

    def slab_body(s, carry):
        start = bnd[0, s]
        end = bnd[0, s + 1]
        rb = lax.rem(s, NR)
        slwait(rb)
        cb_raw = lo + s * SLAB
        is_tail = cb_raw + SLAB > VOCAB_SIZE

        @pl.when(jnp.logical_not(is_tail))
        def _():
            emit_groups(slab_v.at[rb], cb_raw - lo, start, end, SLAB)

        @pl.when(is_tail)
        def _():
            emit_groups(tail_v, (VOCAB_SIZE - 64) - lo, start, end, 64)

        @pl.when(s + NR < NSLAB)
        def _():
            slstage(s + NR)
        return carry

    lax.fori_loop(0, NSLAB, slab_body, 0)


def kernel(x, token_table, position_table):
    xi = x.astype(jnp.int32).reshape(NTOK // 128, 128)
    ttT = token_table.T
    tailT = token_table[VOCAB_SIZE - 64:].T
    run = pl.kernel(
        _emb_body,
        out_type=jax.ShapeDtypeStruct((NTOK + 16, 2 * N_EMBD), jnp.float32),
        mesh=plsc.VectorSubcoreMesh(core_axis_name="c", subcore_axis_name="s"),
        scratch_types=[
            pltpu.VMEM((2, 8, 128), jnp.int32),
            pltpu.VMEM((CAP,), jnp.int32),
            pltpu.VMEM((CAP,), jnp.int32),
            pltpu.VMEM((CAP,), jnp.int32),
            pltpu.VMEM((CAP,), jnp.int32),
            pltpu.VMEM((4, N_EMBD, SLAB), jnp.float32),
            pltpu.VMEM((SEQ_LEN, N_EMBD), jnp.float32),
            pltpu.VMEM((N_EMBD, 64), jnp.float32),
            pltpu.VMEM((4, 16, 2 * N_EMBD), jnp.float32),
            pltpu.VMEM((4, 16), jnp.int32),
            pltpu.SMEM((2, 520), jnp.int32),
            pltpu.SemaphoreType.DMA((2,)),
            pltpu.SemaphoreType.DMA((4,)),
            pltpu.SemaphoreType.DMA((4,)),
        ],
        compiler_params=pltpu.CompilerParams(use_tc_tiling_on_sc=True,
                                             needs_layout_passes=False),
    )
    out3 = run(xi, ttT, position_table, tailT)
    return out3[:NTOK, :N_EMBD].reshape(BATCH, SEQ_LEN, N_EMBD)


# speedup vs baseline: 3.8716x; 3.8716x over previous
import jax
import jax.numpy as jnp
from jax import lax
from jax.experimental import pallas as pl
from jax.experimental.pallas import tpu as pltpu
from jax.experimental.pallas import tpu_sc as plsc

VOCAB_SIZE = 1_000_000
N_EMBD = 64
SEQ_LEN = 200
BATCH = 1024

_info = plsc.get_sparse_core_info()
_NC, _NS = _info.num_cores, _info.num_subcores
NW = _NC * _NS
SEQ_PER_W = BATCH // NW
HALF = SEQ_LEN // 2
NB = 4


def _emb_body(x_hbm, tok_hbm, pos_hbm, out_hbm, idx_v, pos_v, gbuf, sbuf,
              gsems, ssems):
    cid = lax.axis_index("c")
    sid = lax.axis_index("s")
    wid = sid * _NC + cid

    pltpu.sync_copy(pos_hbm, pos_v)
    pltpu.sync_copy(x_hbm.at[pl.ds(wid * SEQ_PER_W, SEQ_PER_W)], idx_v)

    def gather(s, b):
        pltpu.async_copy(tok_hbm.at[idx_v.at[s, 0]],
                         gbuf.at[b, pl.ds(0, HALF)], gsems.at[b])
        pltpu.async_copy(tok_hbm.at[idx_v.at[s, 1]],
                         gbuf.at[b, pl.ds(HALF, HALF)], gsems.at[b])

    def wait_gather(b):
        pltpu.make_async_copy(tok_hbm.at[idx_v.at[0, 0]],
                              gbuf.at[b, pl.ds(0, HALF)], gsems.at[b]).wait()
        pltpu.make_async_copy(tok_hbm.at[idx_v.at[0, 1]],
                              gbuf.at[b, pl.ds(HALF, HALF)], gsems.at[b]).wait()

    def store(s, b):
        out_base = (wid * SEQ_PER_W + s) * SEQ_LEN
        pltpu.async_copy(sbuf.at[b], out_hbm.at[pl.ds(out_base, SEQ_LEN)],
                         ssems.at[b])

    def wait_store(b):
        pltpu.make_async_copy(sbuf.at[b], out_hbm.at[pl.ds(0, SEQ_LEN)],
                              ssems.at[b]).wait()

    for b in range(NB):
        gather(b, b)

    def stage(i, carry):
        for b in range(NB):
            s = i * NB + b
            wait_gather(b)

            @pl.when(s >= NB)
            def _():
                wait_store(b)

            def add_body(r, c):
                base = r * 8
                for k in range(8):
                    rr = base + k
                    for blk in range(N_EMBD // 16):
                        sl = pl.ds(blk * 16, 16)
                        sbuf[b, rr, sl] = gbuf[b, rr, sl] + pos_v[rr, sl]
                return c

            lax.fori_loop(0, SEQ_LEN // 8, add_body, 0)
            store(s, b)

            @pl.when(s + NB < SEQ_PER_W)
            def _():
                gather(s + NB, b)
        return carry

    lax.fori_loop(0, SEQ_PER_W // NB, stage, 0)

    for b in range(NB):
        wait_store(b)


def kernel(x, token_table, position_table):
    x3 = x.reshape(BATCH, 2, HALF).astype(jnp.int32)
    run = pl.kernel(
        _emb_body,
        out_type=jax.ShapeDtypeStruct((BATCH * SEQ_LEN, N_EMBD), jnp.float32),
        mesh=plsc.VectorSubcoreMesh(core_axis_name="c", subcore_axis_name="s"),
        scratch_types=[
            pltpu.VMEM((SEQ_PER_W, 2, HALF), jnp.int32),
            pltpu.VMEM((SEQ_LEN, N_EMBD), jnp.float32),
            pltpu.VMEM((NB, SEQ_LEN, N_EMBD), jnp.float32),
            pltpu.VMEM((NB, SEQ_LEN, N_EMBD), jnp.float32),
            pltpu.SemaphoreType.DMA((NB,)),
            pltpu.SemaphoreType.DMA((NB,)),
        ],
        compiler_params=pltpu.CompilerParams(use_tc_tiling_on_sc=False),
    )
    out = run(x3, token_table, position_table)
    return out.reshape(BATCH, SEQ_LEN, N_EMBD)
